# baseline (speedup 1.0000x reference)
import functools

import jax
import jax.numpy as jnp
from jax import lax
from jax.experimental import pallas as pl
from jax.experimental.pallas import tpu as pltpu
from jax.experimental.pallas import tpu_sc as plsc

E = 320000
NIN = 128
NSEG = 10000
SEGP = 10240
NW = 32

S0 = 163840
S1 = E - S0
BE0 = 16384
BE1 = 15616


def _embed_body(f_ref, w_ref, b_ref, h0_ref, h1_ref):
    f = f_ref[...]
    w = w_ref[...]
    acc = lax.dot_general(w, f, (((1,), (1,)), ((), ())),
                          preferred_element_type=jnp.float32)
    s = acc + b_ref[...][:, 0:1]
    h0_ref[...] = jnp.maximum(s[0:1, :] + s[1:2, :], 0.0)
    h1_ref[...] = jnp.maximum(s[2:3, :] + s[3:4, :], 0.0)


def _embed(features, w8, b8, n, be):
    grid = n // be
    return pl.pallas_call(
        _embed_body,
        grid=(grid,),
        in_specs=[
            pl.BlockSpec((be, NIN), lambda i: (i, 0)),
            pl.BlockSpec((8, NIN), lambda i: (0, 0)),
            pl.BlockSpec((8, NIN), lambda i: (0, 0)),
        ],
        out_specs=[
            pl.BlockSpec((1, be), lambda i: (0, i)),
            pl.BlockSpec((1, be), lambda i: (0, i)),
        ],
        out_shape=[
            jax.ShapeDtypeStruct((1, n), jnp.float32),
            jax.ShapeDtypeStruct((1, n), jnp.float32),
        ],
    )(features, w8, b8)


def _make_seg_body(epw):
    chunks = epw // 16

    def _seg_body(h0_hbm, h1_hbm, idx_hbm, out_hbm, idx_v, h0_v, h1_v, acc_v):
        wid = lax.axis_index("s") * 2 + lax.axis_index("c")
        base = wid * epw
        pltpu.sync_copy(idx_hbm.at[pl.ds(base, epw)], idx_v)
        pltpu.sync_copy(h0_hbm.at[pl.ds(base, epw)], h0_v)
        pltpu.sync_copy(h1_hbm.at[pl.ds(base, epw)], h1_v)

        zero16 = jnp.zeros((16,), jnp.float32)

        def zbody(i, _):
            o = i * 64
            for k in range(4):
                acc_v[0, pl.ds(o + k * 16, 16)] = zero16
                acc_v[1, pl.ds(o + k * 16, 16)] = zero16
                acc_v[2, pl.ds(o + k * 16, 16)] = zero16
            return 0

        lax.fori_loop(0, SEGP // 64, zbody, 0)

        lane = lax.iota(jnp.int32, 16)
        permp = jnp.maximum(lane - 1, 0)
        permn = jnp.minimum(lane + 1, 15)
        lanef = lane.astype(jnp.float32)
        onesc = lanef + 1.0
        row0 = jnp.zeros((16,), jnp.int32)
        row1 = row0 + 1
        row2 = row0 + 2

        def body(j, _):
            o = j * 16
            idx16 = idx_v[pl.ds(o, 16)]
            idxp = plsc.load_gather(idx_v, [o + permp])
            idxn = plsc.load_gather(idx_v, [o + permn])
            first = (idx16 != idxp) | (lane == 0)
            last = (idx16 != idxn) | (lane == 15)
            v0 = h0_v[pl.ds(o, 16)]
            v1 = h1_v[pl.ds(o, 16)]
            cs0 = plsc.cumsum(v0)
            cs1 = plsc.cumsum(v1)
            plsc.addupdate_scatter(acc_v, [row0, idx16], cs0, mask=last)
            plsc.addupdate_scatter(acc_v, [row1, idx16], cs1, mask=last)
            plsc.addupdate_scatter(acc_v, [row2, idx16], onesc, mask=last)
            plsc.addupdate_scatter(acc_v, [row0, idx16], v0 - cs0, mask=first)
            plsc.addupdate_scatter(acc_v, [row1, idx16], v1 - cs1, mask=first)
            plsc.addupdate_scatter(acc_v, [row2, idx16], -lanef, mask=first)
            return 0

        lax.fori_loop(0, chunks, body, 0)
        pltpu.sync_copy(acc_v, out_hbm.at[wid])

    return _seg_body


@functools.cache
def _seg_kernel_fn(n_edges):
    epw = n_edges // NW
    mesh = plsc.VectorSubcoreMesh(
        core_axis_name="c", subcore_axis_name="s",
        num_cores=2, num_subcores=16)
    return pl.kernel(
        _make_seg_body(epw),
        out_type=jax.ShapeDtypeStruct((NW, 3, SEGP), jnp.float32),
        mesh=mesh,
        compiler_params=pltpu.CompilerParams(needs_layout_passes=False),
        scratch_types=[
            pltpu.VMEM((epw,), jnp.int32),
            pltpu.VMEM((epw,), jnp.float32),
            pltpu.VMEM((epw,), jnp.float32),
            pltpu.VMEM((3, SEGP), jnp.float32),
        ],
    )


def _mix_body(pa_ref, pb_ref, w3_ref, w23_ref, s_ref, w4_ref, out_ref):
    red = jnp.sum(pa_ref[...], axis=0) + jnp.sum(pb_ref[...], axis=0)
    h8 = jnp.concatenate(
        [red, jnp.ones((1, SEGP), jnp.float32),
         jnp.zeros((4, SEGP), jnp.float32)], axis=0)
    mc = lax.dot_general(w3_ref[...], w23_ref[...], (((1,), (1,)), ((), ())),
                         preferred_element_type=jnp.float32,
                         precision=lax.Precision.HIGHEST)
    a8 = mc + s_ref[...]
    g = jnp.maximum(
        lax.dot_general(a8, h8, (((1,), (0,)), ((), ())),
                        preferred_element_type=jnp.float32,
                        precision=lax.Precision.HIGHEST), 0.0)
    out_ref[...] = lax.dot_general(w4_ref[...], g, (((1,), (0,)), ((), ())),
                                   preferred_element_type=jnp.float32,
                                   precision=lax.Precision.HIGHEST)


def _mix(pa, pb, w3p, p23, s8, w4m):
    return pl.pallas_call(
        _mix_body,
        grid=(1,),
        in_specs=[
            pl.BlockSpec((NW, 3, SEGP), lambda i: (0, 0, 0)),
            pl.BlockSpec((NW, 3, SEGP), lambda i: (0, 0, 0)),
            pl.BlockSpec((8, NIN), lambda i: (0, 0)),
            pl.BlockSpec((8, NIN), lambda i: (0, 0)),
            pl.BlockSpec((8, 8), lambda i: (0, 0)),
            pl.BlockSpec((8, 8), lambda i: (0, 0)),
        ],
        out_specs=pl.BlockSpec((8, SEGP), lambda i: (0, 0)),
        out_shape=jax.ShapeDtypeStruct((8, SEGP), jnp.float32),
    )(pa, pb, w3p, p23, s8, w4m)


def kernel(features, indices, W1, b1, W2, b2, W3, b3, W4, b4):
    f32 = jnp.float32
    w1hi = W1.astype(jnp.bfloat16).astype(f32)
    w1lo = W1 - w1hi
    w8 = jnp.zeros((8, NIN), f32)
    w8 = w8.at[0, :].set(w1hi[0]).at[1, :].set(w1lo[0])
    w8 = w8.at[2, :].set(w1hi[1]).at[3, :].set(w1lo[1])
    b8 = jnp.zeros((8, NIN), f32).at[0, 0].set(b1[0]).at[2, 0].set(b1[1])

    f0 = lax.slice_in_dim(features, 0, S0, axis=0)
    f1 = lax.slice_in_dim(features, S0, E, axis=0)
    i0 = lax.slice_in_dim(indices, 0, S0, axis=0)
    i1 = lax.slice_in_dim(indices, S0, E, axis=0)

    h0a, h1a = _embed(f0, w8, b8, S0, BE0)
    pa = _seg_kernel_fn(S0)(h0a.reshape(S0), h1a.reshape(S0), i0)
    h0b, h1b = _embed(f1, w8, b8, S1, BE1)
    pb = _seg_kernel_fn(S1)(h0b.reshape(S1), h1b.reshape(S1), i1)

    w3p = jnp.zeros((8, NIN), f32).at[0:2, :].set(W3)
    p23 = jnp.zeros((8, NIN), f32)
    p23 = p23.at[0, :].set(W2[:, 0]).at[1, :].set(W2[:, 1]).at[2, :].set(b2)
    s8 = jnp.zeros((8, 8), f32).at[0, 3].set(b3[0]).at[1, 3].set(b3[1])
    s8 = s8.at[2, 3].set(1.0)
    w4m = jnp.zeros((8, 8), f32)
    w4m = w4m.at[0, 0].set(W4[0, 0]).at[0, 1].set(W4[0, 1]).at[0, 2].set(b4[0])

    out = _mix(pa, pb, w3p, p23, s8, w4m)
    return out[0, :NSEG].reshape(NSEG, 1)


# speedup vs baseline: 1.8203x; 1.8203x over previous
import functools

import jax
import jax.numpy as jnp
from jax import lax
from jax.experimental import pallas as pl
from jax.experimental.pallas import tpu as pltpu
from jax.experimental.pallas import tpu_sc as plsc

E = 320000
NIN = 128
NSEG = 10000
SEGP = 10240
NW = 32

S0 = 166400
S1 = E - S0
BE = 12800


def _embed_body(f_ref, w_ref, b_ref, h0_ref, h1_ref):
    f = f_ref[...]
    w = w_ref[...]
    acc = lax.dot_general(w, f, (((1,), (1,)), ((), ())),
                          preferred_element_type=jnp.float32)
    s = acc + b_ref[...][:, 0:1]
    h0_ref[...] = jnp.maximum(s[0:1, :] + s[1:2, :], 0.0)
    h1_ref[...] = jnp.maximum(s[2:3, :] + s[3:4, :], 0.0)


def _embed(features, w8, b8, n, off_blocks):
    grid = n // BE
    return pl.pallas_call(
        _embed_body,
        grid=(grid,),
        in_specs=[
            pl.BlockSpec((BE, NIN), lambda i: (off_blocks + i, 0)),
            pl.BlockSpec((8, NIN), lambda i: (0, 0)),
            pl.BlockSpec((8, NIN), lambda i: (0, 0)),
        ],
        out_specs=[
            pl.BlockSpec((1, BE), lambda i: (0, i)),
            pl.BlockSpec((1, BE), lambda i: (0, i)),
        ],
        out_shape=[
            jax.ShapeDtypeStruct((1, n), jnp.float32),
            jax.ShapeDtypeStruct((1, n), jnp.float32),
        ],
    )(features, w8, b8)


def _make_seg_body(epw, idx_off):
    chunks = epw // 16

    def _seg_body(h0_hbm, h1_hbm, idx_hbm, out_hbm, idx_v, h0_v, h1_v, acc_v):
        wid = lax.axis_index("s") * 2 + lax.axis_index("c")
        base = wid * epw
        pltpu.sync_copy(idx_hbm.at[pl.ds(idx_off + base, epw)], idx_v)
        pltpu.sync_copy(h0_hbm.at[pl.ds(base, epw)], h0_v)
        pltpu.sync_copy(h1_hbm.at[pl.ds(base, epw)], h1_v)

        zero16 = jnp.zeros((16,), jnp.float32)

        def zbody(i, _):
            o = i * 64
            for k in range(4):
                acc_v[0, pl.ds(o + k * 16, 16)] = zero16
                acc_v[1, pl.ds(o + k * 16, 16)] = zero16
                acc_v[2, pl.ds(o + k * 16, 16)] = zero16
            return 0

        lax.fori_loop(0, SEGP // 64, zbody, 0)

        lane = lax.iota(jnp.int32, 16)
        permp = jnp.maximum(lane - 1, 0)
        permn = jnp.minimum(lane + 1, 15)
        lanef = lane.astype(jnp.float32)
        onesc = lanef + 1.0
        row0 = jnp.zeros((16,), jnp.int32)
        row1 = row0 + 1
        row2 = row0 + 2

        def body(j, _):
            o = j * 16
            idx16 = idx_v[pl.ds(o, 16)]
            idxp = plsc.load_gather(idx_v, [o + permp])
            idxn = plsc.load_gather(idx_v, [o + permn])
            first = (idx16 != idxp) | (lane == 0)
            last = (idx16 != idxn) | (lane == 15)
            v0 = h0_v[pl.ds(o, 16)]
            v1 = h1_v[pl.ds(o, 16)]
            cs0 = plsc.cumsum(v0)
            cs1 = plsc.cumsum(v1)
            plsc.addupdate_scatter(acc_v, [row0, idx16], cs0, mask=last)
            plsc.addupdate_scatter(acc_v, [row1, idx16], cs1, mask=last)
            plsc.addupdate_scatter(acc_v, [row2, idx16], onesc, mask=last)
            plsc.addupdate_scatter(acc_v, [row0, idx16], v0 - cs0, mask=first)
            plsc.addupdate_scatter(acc_v, [row1, idx16], v1 - cs1, mask=first)
            plsc.addupdate_scatter(acc_v, [row2, idx16], -lanef, mask=first)
            return 0

        lax.fori_loop(0, chunks, body, 0)
        pltpu.sync_copy(acc_v, out_hbm.at[wid])

    return _seg_body


@functools.cache
def _seg_kernel_fn(n_edges, idx_off):
    epw = n_edges // NW
    mesh = plsc.VectorSubcoreMesh(
        core_axis_name="c", subcore_axis_name="s",
        num_cores=2, num_subcores=16)
    return pl.kernel(
        _make_seg_body(epw, idx_off),
        out_type=jax.ShapeDtypeStruct((NW, 3, SEGP), jnp.float32),
        mesh=mesh,
        compiler_params=pltpu.CompilerParams(needs_layout_passes=False),
        scratch_types=[
            pltpu.VMEM((epw,), jnp.int32),
            pltpu.VMEM((epw,), jnp.float32),
            pltpu.VMEM((epw,), jnp.float32),
            pltpu.VMEM((3, SEGP), jnp.float32),
        ],
    )


def _mix_body(pa_ref, pb_ref, w3_ref, w23_ref, s_ref, w4_ref, out_ref):
    red = jnp.sum(pa_ref[...], axis=0) + jnp.sum(pb_ref[...], axis=0)
    h8 = jnp.concatenate(
        [red, jnp.ones((1, SEGP), jnp.float32),
         jnp.zeros((4, SEGP), jnp.float32)], axis=0)
    mc = lax.dot_general(w3_ref[...], w23_ref[...], (((1,), (1,)), ((), ())),
                         preferred_element_type=jnp.float32,
                         precision=lax.Precision.HIGHEST)
    a8 = mc + s_ref[...]
    g = jnp.maximum(
        lax.dot_general(a8, h8, (((1,), (0,)), ((), ())),
                        preferred_element_type=jnp.float32,
                        precision=lax.Precision.HIGHEST), 0.0)
    out_ref[...] = lax.dot_general(w4_ref[...], g, (((1,), (0,)), ((), ())),
                                   preferred_element_type=jnp.float32,
                                   precision=lax.Precision.HIGHEST)


def _mix(pa, pb, w3p, p23, s8, w4m):
    return pl.pallas_call(
        _mix_body,
        grid=(1,),
        in_specs=[
            pl.BlockSpec((NW, 3, SEGP), lambda i: (0, 0, 0)),
            pl.BlockSpec((NW, 3, SEGP), lambda i: (0, 0, 0)),
            pl.BlockSpec((8, NIN), lambda i: (0, 0)),
            pl.BlockSpec((8, NIN), lambda i: (0, 0)),
            pl.BlockSpec((8, 8), lambda i: (0, 0)),
            pl.BlockSpec((8, 8), lambda i: (0, 0)),
        ],
        out_specs=pl.BlockSpec((8, SEGP), lambda i: (0, 0)),
        out_shape=jax.ShapeDtypeStruct((8, SEGP), jnp.float32),
    )(pa, pb, w3p, p23, s8, w4m)


def kernel(features, indices, W1, b1, W2, b2, W3, b3, W4, b4):
    f32 = jnp.float32
    w1hi = W1.astype(jnp.bfloat16).astype(f32)
    w1lo = W1 - w1hi
    w8 = jnp.zeros((8, NIN), f32)
    w8 = w8.at[0, :].set(w1hi[0]).at[1, :].set(w1lo[0])
    w8 = w8.at[2, :].set(w1hi[1]).at[3, :].set(w1lo[1])
    b8 = jnp.zeros((8, NIN), f32).at[0, 0].set(b1[0]).at[2, 0].set(b1[1])

    h0a, h1a = _embed(features, w8, b8, S0, 0)
    pa = _seg_kernel_fn(S0, 0)(h0a.reshape(S0), h1a.reshape(S0), indices)
    h0b, h1b = _embed(features, w8, b8, S1, S0 // BE)
    pb = _seg_kernel_fn(S1, S0)(h0b.reshape(S1), h1b.reshape(S1), indices)

    w3p = jnp.zeros((8, NIN), f32).at[0:2, :].set(W3)
    p23 = jnp.zeros((8, NIN), f32)
    p23 = p23.at[0, :].set(W2[:, 0]).at[1, :].set(W2[:, 1]).at[2, :].set(b2)
    s8 = jnp.zeros((8, 8), f32).at[0, 3].set(b3[0]).at[1, 3].set(b3[1])
    s8 = s8.at[2, 3].set(1.0)
    w4m = jnp.zeros((8, 8), f32)
    w4m = w4m.at[0, 0].set(W4[0, 0]).at[0, 1].set(W4[0, 1]).at[0, 2].set(b4[0])

    out = _mix(pa, pb, w3p, p23, s8, w4m)
    return out[0, :NSEG].reshape(NSEG, 1)
